# initial kernel scaffold (unmeasured)
import functools

import jax
import jax.numpy as jnp
from jax import lax
from jax.experimental import pallas as pl
from jax.experimental.pallas import tpu as pltpu

N_DEV = 4
SQ = 2048
SKV = 2048
HQ = 8
DH = 128
DM = 1024
BLK = 64
SCALE = 0.08838834764831843
QC = 512
NQC = SQ // QC

_sem_signal = getattr(pl, "semaphore_signal", None) or pltpu.semaphore_signal
_sem_wait = getattr(pl, "semaphore_wait", None) or pltpu.semaphore_wait
_CompilerParams = getattr(pltpu, "CompilerParams", None) or pltpu.TPUCompilerParams


def kernel(x, Wq, K_ext, V_ext, Wo):
    xb = x.astype(jnp.bfloat16)
    wqb = Wq.astype(jnp.bfloat16)
    wob = Wo.astype(jnp.bfloat16)
    kb = K_ext.astype(jnp.bfloat16)
    vb = V_ext.astype(jnp.bfloat16)

    def body(x_ref, wq_ref, k_ref, v_ref, wo_ref, out_ref,
             kv_buf, q_buf, ctx_buf, comm_buf,
             scat_send_sems, scat_recv_sems, copy_sems,
             ag_send_sems, ag_recv_sems):
        my = lax.axis_index("i")

        bar = pltpu.get_barrier_semaphore()
        for d in range(N_DEV):
            @pl.when(my != d)
            def _():
                _sem_signal(bar, inc=1, device_id=(d,),
                            device_id_type=pl.DeviceIdType.MESH)
        _sem_wait(bar, N_DEV - 1)

        def kvref(t):
            return k_ref if t == 0 else v_ref

        def scat_desc(j, t, h):
            return pltpu.make_async_remote_copy(
                src_ref=kvref(t).at[0, :, 8 * j + h, :],
                dst_ref=kv_buf.at[t, h],
                send_sem=scat_send_sems.at[j - 1, t],
                recv_sem=scat_recv_sems.at[t],
                device_id=(j,),
                device_id_type=pl.DeviceIdType.MESH,
            )

        def local_desc(t, h):
            return pltpu.make_async_copy(
                kvref(t).at[0, :, h, :], kv_buf.at[t, h], copy_sems.at[t],
            )

        @pl.when(my == 0)
        def _():
            for j in range(1, N_DEV):
                for t in range(2):
                    for h in range(HQ):
                        scat_desc(j, t, h).start()
            for t in range(2):
                for h in range(HQ):
                    local_desc(t, h).start()

        q = lax.dot_general(
            x_ref[0], wq_ref[...],
            (((1,), (0,)), ((), ())),
            preferred_element_type=jnp.float32,
        )
        q_buf[...] = (q * SCALE).astype(jnp.bfloat16)

        @pl.when(my == 0)
        def _():
            for t in range(2):
                for h in range(HQ):
                    local_desc(t, h).wait()
            for j in range(1, N_DEV):
                for t in range(2):
                    for h in range(HQ):
                        scat_desc(j, t, h).wait_send()

        @pl.when(my != 0)
        def _():
            for t in range(2):
                for h in range(HQ):
                    scat_desc(1, t, h).wait_recv()

        for h in range(HQ):
            for c in range(NQC):
                kmax = QC * (c + 1)
                qc = q_buf[pl.ds(QC * c, QC), pl.ds(DH * h, DH)]
                kh = kv_buf[0, h, pl.ds(0, kmax), :]
                vh = kv_buf[1, h, pl.ds(0, kmax), :]
                s = lax.dot_general(
                    qc, kh, (((1,), (1,)), ((), ())),
                    preferred_element_type=jnp.float32,
                )
                row = lax.broadcasted_iota(jnp.int32, (QC, kmax), 0) + QC * c
                col = lax.broadcasted_iota(jnp.int32, (QC, kmax), 1)
                s = jnp.where((col // BLK) <= (row // BLK), s, -1e9)
                m = jnp.max(s, axis=1, keepdims=True)
                w = jnp.exp(s - m)
                p = (w / jnp.sum(w, axis=1, keepdims=True)).astype(jnp.bfloat16)
                ctx = lax.dot_general(
                    p, vh, (((1,), (0,)), ((), ())),
                    preferred_element_type=jnp.float32,
                )
                ctx_buf[pl.ds(QC * c, QC), pl.ds(DH * h, DH)] = (
                    ctx.astype(jnp.bfloat16)
                )

        part = lax.dot_general(
            ctx_buf[...], wo_ref[...], (((1,), (0,)), ((), ())),
            preferred_element_type=jnp.float32,
        )
        out_ref[0, :, :] = part
        comm_buf[0, :, :] = part.astype(jnp.bfloat16)

        right = lax.rem(my + 1, N_DEV)
        for hop in range(N_DEV - 1):
            rdma = pltpu.make_async_remote_copy(
                src_ref=comm_buf.at[hop],
                dst_ref=comm_buf.at[hop + 1],
                send_sem=ag_send_sems.at[hop],
                recv_sem=ag_recv_sems.at[hop],
                device_id=(right,),
                device_id_type=pl.DeviceIdType.MESH,
            )
            rdma.start()
            rdma.wait()
            out_ref[0, :, :] = out_ref[0, :, :] + comm_buf[
                hop + 1].astype(jnp.float32)

        @functools.partial(pl.run_scoped, sem2=pltpu.SemaphoreType.REGULAR)
        def _(sem2):
            for d in range(N_DEV):
                @pl.when(my != d)
                def _():
                    _sem_signal(sem2, inc=1, device_id=(d,),
                                device_id_type=pl.DeviceIdType.MESH)
            _sem_wait(sem2, N_DEV - 1)

    return pl.pallas_call(
        body,
        out_shape=jax.ShapeDtypeStruct((1, SQ, DM), jnp.float32),
        in_specs=[
            pl.BlockSpec(memory_space=pltpu.VMEM),
            pl.BlockSpec(memory_space=pltpu.VMEM),
            pl.BlockSpec(memory_space=pltpu.ANY),
            pl.BlockSpec(memory_space=pltpu.ANY),
            pl.BlockSpec(memory_space=pltpu.VMEM),
        ],
        out_specs=pl.BlockSpec(memory_space=pltpu.VMEM),
        scratch_shapes=[
            pltpu.VMEM((2, HQ, SKV, DH), jnp.bfloat16),
            pltpu.VMEM((SQ, DM), jnp.bfloat16),
            pltpu.VMEM((SQ, DM), jnp.bfloat16),
            pltpu.VMEM((N_DEV, SQ, DM), jnp.bfloat16),
            pltpu.SemaphoreType.DMA((N_DEV - 1, 2)),
            pltpu.SemaphoreType.DMA((2,)),
            pltpu.SemaphoreType.DMA((2,)),
            pltpu.SemaphoreType.DMA((N_DEV - 1,)),
            pltpu.SemaphoreType.DMA((N_DEV - 1,)),
        ],
        compiler_params=_CompilerParams(collective_id=0),
    )(xb, wqb, kb, vb, wob)


# baseline (device time: 451959 ns/iter reference)
import functools

import jax
import jax.numpy as jnp
from jax import lax
from jax.experimental import pallas as pl
from jax.experimental.pallas import tpu as pltpu

N_DEV = 4
SQ = 2048
SKV = 2048
HQ = 8
DH = 128
DM = 1024
BLK = 64
SCALE = 0.08838834764831843
QC = 512
NQC = SQ // QC

_sem_signal = getattr(pl, "semaphore_signal", None) or pltpu.semaphore_signal
_sem_wait = getattr(pl, "semaphore_wait", None) or pltpu.semaphore_wait
_CompilerParams = getattr(pltpu, "CompilerParams", None) or pltpu.TPUCompilerParams


def kernel(x, Wq, K_ext, V_ext, Wo):
    xb = x.astype(jnp.bfloat16)
    wqb = Wq.astype(jnp.bfloat16)
    wob = Wo.astype(jnp.bfloat16)
    kb = K_ext.astype(jnp.bfloat16).reshape(1, SKV, 32 * DH)
    vb = V_ext.astype(jnp.bfloat16).reshape(1, SKV, 32 * DH)

    def body(x_ref, wq_ref, k_ref, v_ref, wo_ref, out_ref,
             kv_buf, q_buf, ctx_buf, comm_buf,
             scat_send_sems, scat_recv_sems, copy_sems,
             ag_send_sems, ag_recv_sems):
        my = lax.axis_index("i")

        bar = pltpu.get_barrier_semaphore()
        for d in range(N_DEV):
            @pl.when(my != d)
            def _():
                _sem_signal(bar, inc=1, device_id=(d,),
                            device_id_type=pl.DeviceIdType.MESH)
        _sem_wait(bar, N_DEV - 1)

        def kvref(t):
            return k_ref if t == 0 else v_ref

        def scat_desc(j, t):
            return pltpu.make_async_remote_copy(
                src_ref=kvref(t).at[0, :, pl.ds(DM * j, DM)],
                dst_ref=kv_buf.at[t],
                send_sem=scat_send_sems.at[j - 1, t],
                recv_sem=scat_recv_sems.at[t],
                device_id=(j,),
                device_id_type=pl.DeviceIdType.MESH,
            )

        def local_desc(t):
            return pltpu.make_async_copy(
                kvref(t).at[0, :, pl.ds(0, DM)], kv_buf.at[t], copy_sems.at[t],
            )

        @pl.when(my == 0)
        def _():
            for j in range(1, N_DEV):
                for t in range(2):
                    scat_desc(j, t).start()
            for t in range(2):
                local_desc(t).start()

        q = lax.dot_general(
            x_ref[0], wq_ref[...],
            (((1,), (0,)), ((), ())),
            preferred_element_type=jnp.float32,
        )
        q_buf[...] = (q * SCALE).astype(jnp.bfloat16)

        @pl.when(my == 0)
        def _():
            for t in range(2):
                local_desc(t).wait()
            for j in range(1, N_DEV):
                for t in range(2):
                    scat_desc(j, t).wait_send()

        @pl.when(my != 0)
        def _():
            for t in range(2):
                scat_desc(1, t).wait_recv()

        for h in range(HQ):
            for c in range(NQC):
                kmax = QC * (c + 1)
                qc = q_buf[pl.ds(QC * c, QC), pl.ds(DH * h, DH)]
                kh = kv_buf[0, pl.ds(0, kmax), pl.ds(DH * h, DH)]
                vh = kv_buf[1, pl.ds(0, kmax), pl.ds(DH * h, DH)]
                s = lax.dot_general(
                    qc, kh, (((1,), (1,)), ((), ())),
                    preferred_element_type=jnp.float32,
                )
                row = lax.broadcasted_iota(jnp.int32, (QC, kmax), 0) + QC * c
                col = lax.broadcasted_iota(jnp.int32, (QC, kmax), 1)
                s = jnp.where((col // BLK) <= (row // BLK), s, -1e9)
                m = jnp.max(s, axis=1, keepdims=True)
                w = jnp.exp(s - m)
                p = (w / jnp.sum(w, axis=1, keepdims=True)).astype(jnp.bfloat16)
                ctx = lax.dot_general(
                    p, vh, (((1,), (0,)), ((), ())),
                    preferred_element_type=jnp.float32,
                )
                ctx_buf[pl.ds(QC * c, QC), pl.ds(DH * h, DH)] = (
                    ctx.astype(jnp.bfloat16)
                )

        part = lax.dot_general(
            ctx_buf[...], wo_ref[...], (((1,), (0,)), ((), ())),
            preferred_element_type=jnp.float32,
        )
        out_ref[0, :, :] = part
        comm_buf[0, :, :] = part.astype(jnp.bfloat16)

        right = lax.rem(my + 1, N_DEV)
        for hop in range(N_DEV - 1):
            rdma = pltpu.make_async_remote_copy(
                src_ref=comm_buf.at[hop],
                dst_ref=comm_buf.at[hop + 1],
                send_sem=ag_send_sems.at[hop],
                recv_sem=ag_recv_sems.at[hop],
                device_id=(right,),
                device_id_type=pl.DeviceIdType.MESH,
            )
            rdma.start()
            rdma.wait()
            out_ref[0, :, :] = out_ref[0, :, :] + comm_buf[
                hop + 1].astype(jnp.float32)

        @functools.partial(pl.run_scoped, sem2=pltpu.SemaphoreType.REGULAR)
        def _(sem2):
            for d in range(N_DEV):
                @pl.when(my != d)
                def _():
                    _sem_signal(sem2, inc=1, device_id=(d,),
                                device_id_type=pl.DeviceIdType.MESH)
            _sem_wait(sem2, N_DEV - 1)

    return pl.pallas_call(
        body,
        out_shape=jax.ShapeDtypeStruct((1, SQ, DM), jnp.float32),
        in_specs=[
            pl.BlockSpec(memory_space=pltpu.VMEM),
            pl.BlockSpec(memory_space=pltpu.VMEM),
            pl.BlockSpec(memory_space=pl.ANY),
            pl.BlockSpec(memory_space=pl.ANY),
            pl.BlockSpec(memory_space=pltpu.VMEM),
        ],
        out_specs=pl.BlockSpec(memory_space=pltpu.VMEM),
        scratch_shapes=[
            pltpu.VMEM((2, SKV, DM), jnp.bfloat16),
            pltpu.VMEM((SQ, DM), jnp.bfloat16),
            pltpu.VMEM((SQ, DM), jnp.bfloat16),
            pltpu.VMEM((N_DEV, SQ, DM), jnp.bfloat16),
            pltpu.SemaphoreType.DMA((N_DEV - 1, 2)),
            pltpu.SemaphoreType.DMA((2,)),
            pltpu.SemaphoreType.DMA((2,)),
            pltpu.SemaphoreType.DMA((N_DEV - 1,)),
            pltpu.SemaphoreType.DMA((N_DEV - 1,)),
        ],
        compiler_params=_CompilerParams(
            collective_id=0, vmem_limit_bytes=100 * 1024 * 1024,
        ),
    )(xb, wqb, kb, vb, wob)


# device time: 364922 ns/iter; 1.2385x vs baseline; 1.2385x over previous
import functools

import jax
import jax.numpy as jnp
from jax import lax
from jax.experimental import pallas as pl
from jax.experimental.pallas import tpu as pltpu

N_DEV = 4
SQ = 2048
SKV = 2048
HQ = 8
DH = 128
DM = 1024
BLK = 64
SCALE = 0.08838834764831843
QC = 512
NC = SQ // QC
QTR = SQ // N_DEV

_sem_signal = getattr(pl, "semaphore_signal", None) or pltpu.semaphore_signal
_sem_wait = getattr(pl, "semaphore_wait", None) or pltpu.semaphore_wait
_CompilerParams = getattr(pltpu, "CompilerParams", None) or pltpu.TPUCompilerParams


def kernel(x, Wq, K_ext, V_ext, Wo):
    xb = x.astype(jnp.bfloat16)
    wqb = Wq.astype(jnp.bfloat16)
    wob = Wo.astype(jnp.bfloat16)
    kb = K_ext.astype(jnp.bfloat16).reshape(1, SKV, 32 * DH)
    vb = V_ext.astype(jnp.bfloat16).reshape(1, SKV, 32 * DH)

    def body(x_ref, wq_ref, k_ref, v_ref, wo_ref, out_ref,
             kv_buf, q_buf, ctx_buf, part_buf,
             rs_send, rs_recv, ag_send0, ag_recv,
             scat_send_sems, scat_recv_sems, copy_sems,
             rs_send_sems, rs_recv_sems, ag_send_sems, ag_recv_sems):
        my = lax.axis_index("i")
        right = lax.rem(my + 1, N_DEV)

        bar = pltpu.get_barrier_semaphore()
        for d in range(N_DEV):
            @pl.when(my != d)
            def _():
                _sem_signal(bar, inc=1, device_id=(d,),
                            device_id_type=pl.DeviceIdType.MESH)
        _sem_wait(bar, N_DEV - 1)

        def kvref(t):
            return k_ref if t == 0 else v_ref

        def scat_desc(j, t, c):
            return pltpu.make_async_remote_copy(
                src_ref=kvref(t).at[0, pl.ds(QC * c, QC), pl.ds(DM * j, DM)],
                dst_ref=kv_buf.at[t, pl.ds(QC * c, QC), :],
                send_sem=scat_send_sems.at[j - 1, t, c],
                recv_sem=scat_recv_sems.at[t, c],
                device_id=(j,),
                device_id_type=pl.DeviceIdType.MESH,
            )

        def local_desc(t):
            return pltpu.make_async_copy(
                kvref(t).at[0, :, pl.ds(0, DM)], kv_buf.at[t], copy_sems.at[t],
            )

        @pl.when(my == 0)
        def _():
            for c in range(NC):
                for j in range(1, N_DEV):
                    for t in range(2):
                        scat_desc(j, t, c).start()
            for t in range(2):
                local_desc(t).start()

        q = lax.dot_general(
            x_ref[0], wq_ref[...],
            (((1,), (0,)), ((), ())),
            preferred_element_type=jnp.float32,
        )
        q_buf[...] = (q * SCALE).astype(jnp.bfloat16)

        @pl.when(my == 0)
        def _():
            for t in range(2):
                local_desc(t).wait()

        for c in range(NC):
            @pl.when(my != 0)
            def _():
                for t in range(2):
                    scat_desc(1, t, c).wait_recv()

            kmax = QC * (c + 1)
            for h in range(HQ):
                qc = q_buf[pl.ds(QC * c, QC), pl.ds(DH * h, DH)]
                kh = kv_buf[0, pl.ds(0, kmax), pl.ds(DH * h, DH)]
                vh = kv_buf[1, pl.ds(0, kmax), pl.ds(DH * h, DH)]
                s = lax.dot_general(
                    qc, kh, (((1,), (1,)), ((), ())),
                    preferred_element_type=jnp.float32,
                )
                row = lax.broadcasted_iota(jnp.int32, (QC, kmax), 0) + QC * c
                col = lax.broadcasted_iota(jnp.int32, (QC, kmax), 1)
                s = jnp.where((col // BLK) <= (row // BLK), s, -1e9)
                m = jnp.max(s, axis=1, keepdims=True)
                w = jnp.exp(s - m)
                p = (w / jnp.sum(w, axis=1, keepdims=True)).astype(jnp.bfloat16)
                ctx = lax.dot_general(
                    p, vh, (((1,), (0,)), ((), ())),
                    preferred_element_type=jnp.float32,
                )
                ctx_buf[pl.ds(QC * c, QC), pl.ds(DH * h, DH)] = (
                    ctx.astype(jnp.bfloat16)
                )

        part_buf[...] = lax.dot_general(
            ctx_buf[...], wo_ref[...], (((1,), (0,)), ((), ())),
            preferred_element_type=jnp.float32,
        )

        @pl.when(my == 0)
        def _():
            for c in range(NC):
                for j in range(1, N_DEV):
                    for t in range(2):
                        scat_desc(j, t, c).wait_send()

        def qslice(buf, cid):
            return buf[pl.ds(cid * QTR, QTR), :]

        rs_send[0] = qslice(part_buf, my).astype(jnp.bfloat16)
        for s in range(N_DEV - 1):
            rdma = pltpu.make_async_remote_copy(
                src_ref=rs_send.at[s],
                dst_ref=rs_recv.at[s],
                send_sem=rs_send_sems.at[s],
                recv_sem=rs_recv_sems.at[s],
                device_id=(right,),
                device_id_type=pl.DeviceIdType.MESH,
            )
            rdma.start()
            rdma.wait()
            rcid = lax.rem(my + (N_DEV - 1 - s), N_DEV)
            acc = rs_recv[s].astype(jnp.float32) + qslice(part_buf, rcid)
            if s < N_DEV - 2:
                rs_send[s + 1] = acc.astype(jnp.bfloat16)

        own = lax.rem(my + 1, N_DEV)
        out_ref[0, pl.ds(own * QTR, QTR), :] = acc
        ag_send0[...] = acc.astype(jnp.bfloat16)

        for s in range(N_DEV - 1):
            rdma = pltpu.make_async_remote_copy(
                src_ref=ag_send0 if s == 0 else ag_recv.at[s - 1],
                dst_ref=ag_recv.at[s],
                send_sem=ag_send_sems.at[s],
                recv_sem=ag_recv_sems.at[s],
                device_id=(right,),
                device_id_type=pl.DeviceIdType.MESH,
            )
            rdma.start()
            rdma.wait()
            cid = lax.rem(my + (N_DEV - s), N_DEV)
            out_ref[0, pl.ds(cid * QTR, QTR), :] = (
                ag_recv[s].astype(jnp.float32)
            )

        @functools.partial(pl.run_scoped, sem2=pltpu.SemaphoreType.REGULAR)
        def _(sem2):
            for d in range(N_DEV):
                @pl.when(my != d)
                def _():
                    _sem_signal(sem2, inc=1, device_id=(d,),
                                device_id_type=pl.DeviceIdType.MESH)
            _sem_wait(sem2, N_DEV - 1)

    return pl.pallas_call(
        body,
        out_shape=jax.ShapeDtypeStruct((1, SQ, DM), jnp.float32),
        in_specs=[
            pl.BlockSpec(memory_space=pltpu.VMEM),
            pl.BlockSpec(memory_space=pltpu.VMEM),
            pl.BlockSpec(memory_space=pl.ANY),
            pl.BlockSpec(memory_space=pl.ANY),
            pl.BlockSpec(memory_space=pltpu.VMEM),
        ],
        out_specs=pl.BlockSpec(memory_space=pltpu.VMEM),
        scratch_shapes=[
            pltpu.VMEM((2, SKV, DM), jnp.bfloat16),
            pltpu.VMEM((SQ, DM), jnp.bfloat16),
            pltpu.VMEM((SQ, DM), jnp.bfloat16),
            pltpu.VMEM((SQ, DM), jnp.float32),
            pltpu.VMEM((N_DEV - 1, QTR, DM), jnp.bfloat16),
            pltpu.VMEM((N_DEV - 1, QTR, DM), jnp.bfloat16),
            pltpu.VMEM((QTR, DM), jnp.bfloat16),
            pltpu.VMEM((N_DEV - 1, QTR, DM), jnp.bfloat16),
            pltpu.SemaphoreType.DMA((N_DEV - 1, 2, NC)),
            pltpu.SemaphoreType.DMA((2, NC)),
            pltpu.SemaphoreType.DMA((2,)),
            pltpu.SemaphoreType.DMA((N_DEV - 1,)),
            pltpu.SemaphoreType.DMA((N_DEV - 1,)),
            pltpu.SemaphoreType.DMA((N_DEV - 1,)),
            pltpu.SemaphoreType.DMA((N_DEV - 1,)),
        ],
        compiler_params=_CompilerParams(
            collective_id=0, vmem_limit_bytes=100 * 1024 * 1024,
        ),
    )(xb, wqb, kb, vb, wob)


# device time: 330404 ns/iter; 1.3679x vs baseline; 1.1045x over previous
import functools

import jax
import jax.numpy as jnp
from jax import lax
from jax.experimental import pallas as pl
from jax.experimental.pallas import tpu as pltpu

N_DEV = 4
SQ = 2048
SKV = 2048
HQ = 8
DH = 128
DM = 1024
BLK = 64
SCALE = 0.08838834764831843
QC = 512
NC = SQ // QC
QTR = SQ // N_DEV

_sem_signal = getattr(pl, "semaphore_signal", None) or pltpu.semaphore_signal
_sem_wait = getattr(pl, "semaphore_wait", None) or pltpu.semaphore_wait
_CompilerParams = getattr(pltpu, "CompilerParams", None) or pltpu.TPUCompilerParams


def kernel(x, Wq, K_ext, V_ext, Wo):
    xb = x.astype(jnp.bfloat16)
    wqb = Wq.astype(jnp.bfloat16)
    wob = Wo.astype(jnp.bfloat16)
    kb = K_ext.astype(jnp.bfloat16).reshape(1, SKV, 32 * DH)
    vb = V_ext.astype(jnp.bfloat16).reshape(1, SKV, 32 * DH)

    def body(x_ref, wq_ref, k_ref, v_ref, wo_ref, out_ref,
             kv_buf, q_buf, ctx_buf, part_buf, relay_buf,
             rs_send, rs_recv, ag_send0, ag_recv,
             scat_send_sems, scat_recv_sems, copy_sems,
             relay_recv_sems, fwd_send_sems,
             rs_send_sems, rs_recv_sems, ag_send_sems, ag_recv_sems):
        my = lax.axis_index("i")
        right = lax.rem(my + 1, N_DEV)

        bar = pltpu.get_barrier_semaphore()
        for d in range(N_DEV):
            @pl.when(my != d)
            def _():
                _sem_signal(bar, inc=1, device_id=(d,),
                            device_id_type=pl.DeviceIdType.MESH)
        _sem_wait(bar, N_DEV - 1)

        def kvref(t):
            return k_ref if t == 0 else v_ref

        def scat_desc(j, t, c):
            return pltpu.make_async_remote_copy(
                src_ref=kvref(t).at[0, pl.ds(QC * c, QC), pl.ds(DM * j, DM)],
                dst_ref=kv_buf.at[t, pl.ds(QC * c, QC), :],
                send_sem=scat_send_sems.at[j - 1, t, c],
                recv_sem=scat_recv_sems.at[t, c],
                device_id=(j,),
                device_id_type=pl.DeviceIdType.MESH,
            )

        def relay_in_desc(t, c):
            return pltpu.make_async_remote_copy(
                src_ref=kvref(t).at[0, pl.ds(QC * c, QC), pl.ds(DM * 2, DM)],
                dst_ref=relay_buf.at[pl.ds(QC * c, QC), :],
                send_sem=scat_send_sems.at[1, t, c],
                recv_sem=relay_recv_sems.at[c],
                device_id=(1 if t == 0 else 3,),
                device_id_type=pl.DeviceIdType.MESH,
            )

        def fwd_desc(t, c):
            return pltpu.make_async_remote_copy(
                src_ref=relay_buf.at[pl.ds(QC * c, QC), :],
                dst_ref=kv_buf.at[t, pl.ds(QC * c, QC), :],
                send_sem=fwd_send_sems.at[c],
                recv_sem=scat_recv_sems.at[t, c],
                device_id=(2,),
                device_id_type=pl.DeviceIdType.MESH,
            )

        def local_desc(t):
            return pltpu.make_async_copy(
                kvref(t).at[0, :, pl.ds(0, DM)], kv_buf.at[t], copy_sems.at[t],
            )

        @pl.when(my == 0)
        def _():
            for c in range(NC):
                for t in range(2):
                    scat_desc(1, t, c).start()
                    scat_desc(3, t, c).start()
                    relay_in_desc(t, c).start()
            for t in range(2):
                local_desc(t).start()

        q = lax.dot_general(
            x_ref[0], wq_ref[...],
            (((1,), (0,)), ((), ())),
            preferred_element_type=jnp.float32,
        )
        q_buf[...] = (q * SCALE).astype(jnp.bfloat16)

        @pl.when(my == 0)
        def _():
            for t in range(2):
                local_desc(t).wait()

        for c in range(NC):
            @pl.when(my == 1)
            def _():
                relay_in_desc(0, c).wait_recv()
                fwd_desc(0, c).start()

            @pl.when(my == 3)
            def _():
                relay_in_desc(1, c).wait_recv()
                fwd_desc(1, c).start()

            @pl.when(my != 0)
            def _():
                for t in range(2):
                    scat_desc(1, t, c).wait_recv()

            kmax = QC * (c + 1)
            row = lax.broadcasted_iota(jnp.int32, (QC, kmax), 0) + QC * c
            col = lax.broadcasted_iota(jnp.int32, (QC, kmax), 1)
            bias = jnp.where((col // BLK) <= (row // BLK),
                             jnp.float32(0.0), jnp.float32(-1e9))
            for h in range(HQ):
                qc = q_buf[pl.ds(QC * c, QC), pl.ds(DH * h, DH)]
                kh = kv_buf[0, pl.ds(0, kmax), pl.ds(DH * h, DH)]
                vh = kv_buf[1, pl.ds(0, kmax), pl.ds(DH * h, DH)]
                s = lax.dot_general(
                    qc, kh, (((1,), (1,)), ((), ())),
                    preferred_element_type=jnp.float32,
                ) + bias
                w = jnp.exp(s)
                p = (w * (1.0 / jnp.sum(w, axis=1, keepdims=True))
                     ).astype(jnp.bfloat16)
                ctx = lax.dot_general(
                    p, vh, (((1,), (0,)), ((), ())),
                    preferred_element_type=jnp.float32,
                )
                ctx_buf[pl.ds(QC * c, QC), pl.ds(DH * h, DH)] = (
                    ctx.astype(jnp.bfloat16)
                )

        part_buf[...] = lax.dot_general(
            ctx_buf[...], wo_ref[...], (((1,), (0,)), ((), ())),
            preferred_element_type=jnp.float32,
        )

        @pl.when(my == 0)
        def _():
            for c in range(NC):
                for t in range(2):
                    scat_desc(1, t, c).wait_send()
                    scat_desc(3, t, c).wait_send()
                    relay_in_desc(t, c).wait_send()

        @pl.when(my == 1)
        def _():
            for c in range(NC):
                fwd_desc(0, c).wait_send()

        @pl.when(my == 3)
        def _():
            for c in range(NC):
                fwd_desc(1, c).wait_send()

        def qslice(buf, cid):
            return buf[pl.ds(cid * QTR, QTR), :]

        rs_send[0] = qslice(part_buf, my).astype(jnp.bfloat16)
        for s in range(N_DEV - 1):
            rdma = pltpu.make_async_remote_copy(
                src_ref=rs_send.at[s],
                dst_ref=rs_recv.at[s],
                send_sem=rs_send_sems.at[s],
                recv_sem=rs_recv_sems.at[s],
                device_id=(right,),
                device_id_type=pl.DeviceIdType.MESH,
            )
            rdma.start()
            rdma.wait()
            rcid = lax.rem(my + (N_DEV - 1 - s), N_DEV)
            acc = rs_recv[s].astype(jnp.float32) + qslice(part_buf, rcid)
            if s < N_DEV - 2:
                rs_send[s + 1] = acc.astype(jnp.bfloat16)

        own = lax.rem(my + 1, N_DEV)
        out_ref[0, pl.ds(own * QTR, QTR), :] = acc
        ag_send0[...] = acc.astype(jnp.bfloat16)

        for s in range(N_DEV - 1):
            rdma = pltpu.make_async_remote_copy(
                src_ref=ag_send0 if s == 0 else ag_recv.at[s - 1],
                dst_ref=ag_recv.at[s],
                send_sem=ag_send_sems.at[s],
                recv_sem=ag_recv_sems.at[s],
                device_id=(right,),
                device_id_type=pl.DeviceIdType.MESH,
            )
            rdma.start()
            rdma.wait()
            cid = lax.rem(my + (N_DEV - s), N_DEV)
            out_ref[0, pl.ds(cid * QTR, QTR), :] = (
                ag_recv[s].astype(jnp.float32)
            )

        @functools.partial(pl.run_scoped, sem2=pltpu.SemaphoreType.REGULAR)
        def _(sem2):
            for d in range(N_DEV):
                @pl.when(my != d)
                def _():
                    _sem_signal(sem2, inc=1, device_id=(d,),
                                device_id_type=pl.DeviceIdType.MESH)
            _sem_wait(sem2, N_DEV - 1)

    return pl.pallas_call(
        body,
        out_shape=jax.ShapeDtypeStruct((1, SQ, DM), jnp.float32),
        in_specs=[
            pl.BlockSpec(memory_space=pltpu.VMEM),
            pl.BlockSpec(memory_space=pltpu.VMEM),
            pl.BlockSpec(memory_space=pl.ANY),
            pl.BlockSpec(memory_space=pl.ANY),
            pl.BlockSpec(memory_space=pltpu.VMEM),
        ],
        out_specs=pl.BlockSpec(memory_space=pltpu.VMEM),
        scratch_shapes=[
            pltpu.VMEM((2, SKV, DM), jnp.bfloat16),
            pltpu.VMEM((SQ, DM), jnp.bfloat16),
            pltpu.VMEM((SQ, DM), jnp.bfloat16),
            pltpu.VMEM((SQ, DM), jnp.float32),
            pltpu.VMEM((SKV, DM), jnp.bfloat16),
            pltpu.VMEM((N_DEV - 1, QTR, DM), jnp.bfloat16),
            pltpu.VMEM((N_DEV - 1, QTR, DM), jnp.bfloat16),
            pltpu.VMEM((QTR, DM), jnp.bfloat16),
            pltpu.VMEM((N_DEV - 1, QTR, DM), jnp.bfloat16),
            pltpu.SemaphoreType.DMA((N_DEV - 1, 2, NC)),
            pltpu.SemaphoreType.DMA((2, NC)),
            pltpu.SemaphoreType.DMA((2,)),
            pltpu.SemaphoreType.DMA((NC,)),
            pltpu.SemaphoreType.DMA((NC,)),
            pltpu.SemaphoreType.DMA((N_DEV - 1,)),
            pltpu.SemaphoreType.DMA((N_DEV - 1,)),
            pltpu.SemaphoreType.DMA((N_DEV - 1,)),
            pltpu.SemaphoreType.DMA((N_DEV - 1,)),
        ],
        compiler_params=_CompilerParams(
            collective_id=0, vmem_limit_bytes=100 * 1024 * 1024,
        ),
    )(xb, wqb, kb, vb, wob)


# device time: 294880 ns/iter; 1.5327x vs baseline; 1.1205x over previous
import functools

import jax
import jax.numpy as jnp
from jax import lax
from jax.experimental import pallas as pl
from jax.experimental.pallas import tpu as pltpu

N_DEV = 4
SQ = 2048
SKV = 2048
HQ = 8
DH = 128
DM = 1024
BLK = 64
SCALE = 0.08838834764831843
QC = 512
NC = SQ // QC
QTR = SQ // N_DEV
HALF = QTR // 2

_sem_signal = getattr(pl, "semaphore_signal", None) or pltpu.semaphore_signal
_sem_wait = getattr(pl, "semaphore_wait", None) or pltpu.semaphore_wait
_CompilerParams = getattr(pltpu, "CompilerParams", None) or pltpu.TPUCompilerParams


def kernel(x, Wq, K_ext, V_ext, Wo):
    xb = x.astype(jnp.bfloat16)
    wqb = Wq.astype(jnp.bfloat16)
    wob = Wo.astype(jnp.bfloat16)
    kf = K_ext.reshape(1, SKV, 32 * DH)
    vf = V_ext.reshape(1, SKV, 32 * DH)

    def body(x_ref, wq_ref, k_ref, v_ref, wo_ref, out_ref,
             kv_buf, q_buf, relay_buf, kf_stage, send_stage,
             rs_send, rs_recv, ag_send0, ag_recv,
             scat_send_sems, scat_recv_sems, stage_sems,
             relay_recv_sems, fwd_send_sems,
             rs_send_sems, rs_recv_sems, ag_send_sems, ag_recv_sems):
        my = lax.axis_index("i")
        right = lax.rem(my + 1, N_DEV)
        left = lax.rem(my + 3, N_DEV)

        bar = pltpu.get_barrier_semaphore()
        for d in range(N_DEV):
            @pl.when(my != d)
            def _():
                _sem_signal(bar, inc=1, device_id=(d,),
                            device_id_type=pl.DeviceIdType.MESH)
        _sem_wait(bar, N_DEV - 1)

        def kvref(t):
            return k_ref if t == 0 else v_ref

        def stage_in_desc(t, c):
            return pltpu.make_async_copy(
                kvref(t).at[0, pl.ds(QC * c, QC), :],
                kf_stage,
                stage_sems.at[0],
            )

        def scat_desc(j, t, c):
            col = {1: 0, 3: 2 * DM}[j]
            return pltpu.make_async_remote_copy(
                src_ref=send_stage.at[t, :, pl.ds(col, DM)],
                dst_ref=kv_buf.at[t, pl.ds(QC * c, QC), :],
                send_sem=scat_send_sems.at[j - 1, t, c],
                recv_sem=scat_recv_sems.at[t, c],
                device_id=(j,),
                device_id_type=pl.DeviceIdType.MESH,
            )

        def relay_in_desc(t, c):
            return pltpu.make_async_remote_copy(
                src_ref=send_stage.at[t, :, pl.ds(DM, DM)],
                dst_ref=relay_buf.at[pl.ds(QC * c, QC), :],
                send_sem=scat_send_sems.at[1, t, c],
                recv_sem=relay_recv_sems.at[c],
                device_id=(1 if t == 0 else 3,),
                device_id_type=pl.DeviceIdType.MESH,
            )

        def fwd_desc(t, c):
            return pltpu.make_async_remote_copy(
                src_ref=relay_buf.at[pl.ds(QC * c, QC), :],
                dst_ref=kv_buf.at[t, pl.ds(QC * c, QC), :],
                send_sem=fwd_send_sems.at[c],
                recv_sem=scat_recv_sems.at[t, c],
                device_id=(2,),
                device_id_type=pl.DeviceIdType.MESH,
            )

        def start_sends(t, c):
            relay_in_desc(t, c).start()
            scat_desc(1, t, c).start()
            scat_desc(3, t, c).start()

        def wait_sends(t, c):
            scat_desc(1, t, c).wait_send()
            scat_desc(3, t, c).wait_send()
            relay_in_desc(t, c).wait_send()

        @pl.when(my == 0)
        def _():
            stage_in_desc(0, 0).start()
            for c in range(NC):
                for t in range(2):
                    stage_in_desc(t, c).wait()
                    kv_buf[t, pl.ds(QC * c, QC), :] = (
                        kf_stage[:, pl.ds(0, DM)].astype(jnp.bfloat16)
                    )
                    if c >= 1:
                        wait_sends(t, c - 1)
                    send_stage[t, :, :] = (
                        kf_stage[:, pl.ds(DM, 3 * DM)].astype(jnp.bfloat16)
                    )
                    start_sends(t, c)
                    nt, nc = (1, c) if t == 0 else (0, c + 1)
                    if nc < NC:
                        stage_in_desc(nt, nc).start()

        q = lax.dot_general(
            x_ref[0], wq_ref[...],
            (((1,), (0,)), ((), ())),
            preferred_element_type=jnp.float32,
        )
        q_buf[...] = (q * SCALE).astype(jnp.bfloat16)

        for c in range(NC):
            @pl.when(my == 1)
            def _():
                relay_in_desc(0, c).wait_recv()
                fwd_desc(0, c).start()

            @pl.when(my == 3)
            def _():
                relay_in_desc(1, c).wait_recv()
                fwd_desc(1, c).start()

            @pl.when(my != 0)
            def _():
                for t in range(2):
                    scat_desc(1, t, c).wait_recv()

            kmax = QC * (c + 1)
            row = lax.broadcasted_iota(jnp.int32, (QC, kmax), 0) + QC * c
            col = lax.broadcasted_iota(jnp.int32, (QC, kmax), 1)
            bias = jnp.where((col // BLK) <= (row // BLK),
                             jnp.float32(0.0), jnp.float32(-1e9))
            ctx_heads = []
            for h in range(HQ):
                qc = q_buf[pl.ds(QC * c, QC), pl.ds(DH * h, DH)]
                kh = kv_buf[0, pl.ds(0, kmax), pl.ds(DH * h, DH)]
                vh = kv_buf[1, pl.ds(0, kmax), pl.ds(DH * h, DH)]
                s = lax.dot_general(
                    qc, kh, (((1,), (1,)), ((), ())),
                    preferred_element_type=jnp.float32,
                ) + bias
                w = jnp.exp(s)
                p = (w * (1.0 / jnp.sum(w, axis=1, keepdims=True))
                     ).astype(jnp.bfloat16)
                ctx_heads.append(lax.dot_general(
                    p, vh, (((1,), (0,)), ((), ())),
                    preferred_element_type=jnp.float32,
                ).astype(jnp.bfloat16))

            ctx_c = jnp.concatenate(ctx_heads, axis=1)
            out_ref[0, pl.ds(QC * c, QC), :] = lax.dot_general(
                ctx_c, wo_ref[...], (((1,), (0,)), ((), ())),
                preferred_element_type=jnp.float32,
            ).astype(jnp.bfloat16)

        @pl.when(my == 0)
        def _():
            for t in range(2):
                wait_sends(t, NC - 1)

        @pl.when(my == 1)
        def _():
            for c in range(NC):
                fwd_desc(0, c).wait_send()

        @pl.when(my == 3)
        def _():
            for c in range(NC):
                fwd_desc(1, c).wait_send()

        def pslice(qid, d):
            return out_ref[0, pl.ds(qid * QTR + d * HALF, HALF), :]

        def ring_rdma(src, dst, ssem, rsem, dev):
            return pltpu.make_async_remote_copy(
                src_ref=src, dst_ref=dst, send_sem=ssem, recv_sem=rsem,
                device_id=(dev,), device_id_type=pl.DeviceIdType.MESH,
            )

        rs_send[0, 0] = pslice(my, 0)
        rs_send[1, 0] = pslice(my, 1)
        acc = [None, None]
        for s in range(N_DEV - 1):
            rr = ring_rdma(rs_send.at[0, s], rs_recv.at[0, s],
                           rs_send_sems.at[0, s], rs_recv_sems.at[0, s],
                           right)
            rl = ring_rdma(rs_send.at[1, s], rs_recv.at[1, s],
                           rs_send_sems.at[1, s], rs_recv_sems.at[1, s],
                           left)
            rr.start()
            rl.start()
            rr.wait()
            rl.wait()
            rcid_r = lax.rem(my + (N_DEV - 1 - s), N_DEV)
            rcid_l = lax.rem(my + s + 1, N_DEV)
            acc[0] = rs_recv[0, s].astype(jnp.float32) + pslice(
                rcid_r, 0).astype(jnp.float32)
            acc[1] = rs_recv[1, s].astype(jnp.float32) + pslice(
                rcid_l, 1).astype(jnp.float32)
            if s < N_DEV - 2:
                rs_send[0, s + 1] = acc[0].astype(jnp.bfloat16)
                rs_send[1, s + 1] = acc[1].astype(jnp.bfloat16)

        own_r = lax.rem(my + 1, N_DEV)
        own_l = lax.rem(my + 3, N_DEV)
        red = [acc[0].astype(jnp.bfloat16), acc[1].astype(jnp.bfloat16)]
        out_ref[0, pl.ds(own_r * QTR, HALF), :] = red[0]
        out_ref[0, pl.ds(own_l * QTR + HALF, HALF), :] = red[1]
        ag_send0[0] = red[0]
        ag_send0[1] = red[1]

        for s in range(N_DEV - 1):
            gr = ring_rdma(ag_send0.at[0] if s == 0 else ag_recv.at[0, s - 1],
                           ag_recv.at[0, s],
                           ag_send_sems.at[0, s], ag_recv_sems.at[0, s],
                           right)
            gl = ring_rdma(ag_send0.at[1] if s == 0 else ag_recv.at[1, s - 1],
                           ag_recv.at[1, s],
                           ag_send_sems.at[1, s], ag_recv_sems.at[1, s],
                           left)
            gr.start()
            gl.start()
            gr.wait()
            gl.wait()
            cid_r = lax.rem(my + (N_DEV - s), N_DEV)
            cid_l = lax.rem(my + s, N_DEV)
            out_ref[0, pl.ds(cid_r * QTR, HALF), :] = ag_recv[0, s]
            out_ref[0, pl.ds(cid_l * QTR + HALF, HALF), :] = ag_recv[1, s]

        @functools.partial(pl.run_scoped, sem2=pltpu.SemaphoreType.REGULAR)
        def _(sem2):
            for d in range(N_DEV):
                @pl.when(my != d)
                def _():
                    _sem_signal(sem2, inc=1, device_id=(d,),
                                device_id_type=pl.DeviceIdType.MESH)
            _sem_wait(sem2, N_DEV - 1)

    return pl.pallas_call(
        body,
        out_shape=jax.ShapeDtypeStruct((1, SQ, DM), jnp.bfloat16),
        in_specs=[
            pl.BlockSpec(memory_space=pltpu.VMEM),
            pl.BlockSpec(memory_space=pltpu.VMEM),
            pl.BlockSpec(memory_space=pl.ANY),
            pl.BlockSpec(memory_space=pl.ANY),
            pl.BlockSpec(memory_space=pltpu.VMEM),
        ],
        out_specs=pl.BlockSpec(memory_space=pltpu.VMEM),
        scratch_shapes=[
            pltpu.VMEM((2, SKV, DM), jnp.bfloat16),
            pltpu.VMEM((SQ, DM), jnp.bfloat16),
            pltpu.VMEM((SKV, DM), jnp.bfloat16),
            pltpu.VMEM((QC, 4 * DM), jnp.float32),
            pltpu.VMEM((2, QC, 3 * DM), jnp.bfloat16),
            pltpu.VMEM((2, N_DEV - 1, HALF, DM), jnp.bfloat16),
            pltpu.VMEM((2, N_DEV - 1, HALF, DM), jnp.bfloat16),
            pltpu.VMEM((2, HALF, DM), jnp.bfloat16),
            pltpu.VMEM((2, N_DEV - 1, HALF, DM), jnp.bfloat16),
            pltpu.SemaphoreType.DMA((N_DEV - 1, 2, NC)),
            pltpu.SemaphoreType.DMA((2, NC)),
            pltpu.SemaphoreType.DMA((1,)),
            pltpu.SemaphoreType.DMA((NC,)),
            pltpu.SemaphoreType.DMA((NC,)),
            pltpu.SemaphoreType.DMA((2, N_DEV - 1)),
            pltpu.SemaphoreType.DMA((2, N_DEV - 1)),
            pltpu.SemaphoreType.DMA((2, N_DEV - 1)),
            pltpu.SemaphoreType.DMA((2, N_DEV - 1)),
        ],
        compiler_params=_CompilerParams(
            collective_id=0, vmem_limit_bytes=100 * 1024 * 1024,
        ),
    )(xb, wqb, kf, vf, wob)
